# baseline (device time: 428517 ns/iter reference)
import jax
import jax.numpy as jnp
from jax import lax
from jax.experimental import pallas as pl
from jax.experimental.pallas import tpu as pltpu

N_DEV = 4
SQ = 2048
NB = 32
BS = 64
HC = 1024
NH = 8
DH = 128
NG = 4
GS = SQ // NG
SCALE = 0.08838834764831843


def _group_rows(a):
    s = a.shape[0]
    return a.reshape(8, 4, 64, -1).transpose(1, 0, 2, 3).reshape(s, -1)


def _ungroup_rows(a):
    s = a.shape[0]
    return a.reshape(4, 8, 64, -1).transpose(1, 0, 2, 3).reshape(s, -1)


def _kv_unit(my, aj, g, par, k_hbm, v_hbm, stage_k, stage_v, copy_sems):
    cps = []
    cols = pl.ds(aj * HC, HC)
    for k in range(NH):
        b = NG * k + g
        rows = pl.ds(BS * k, BS)
        cps.append(pltpu.make_async_copy(
            k_hbm.at[my, b, :, cols], stage_k.at[par, rows],
            copy_sems.at[0, par]))
        cps.append(pltpu.make_async_copy(
            v_hbm.at[my, b, :, cols], stage_v.at[par, rows],
            copy_sems.at[1, par]))
    return cps


def _body(x_ref, wq_ref, wo_ref, k_hbm, v_hbm, out_ref,
          wq_comm, wo_comm, stage_k, stage_v, ctx_hold,
          wq_ssem, wq_rsem, wo_ssem, wo_rsem, copy_sems):
    my = lax.axis_index("i")
    left = lax.rem(my + N_DEV - 1, N_DEV)
    right = lax.rem(my + 1, N_DEV)

    barrier_sem = pltpu.get_barrier_semaphore()
    for nbr in (left, right):
        pl.semaphore_signal(barrier_sem, inc=1, device_id=(nbr,),
                            device_id_type=pl.DeviceIdType.MESH)
    pl.semaphore_wait(barrier_sem, 2)

    wq_comm[0] = wq_ref[...]
    wo_comm[0] = wo_ref[...]

    def unit_args(t):
        h, g = t // NG, t % NG
        aj = lax.rem(my + N_DEV - h, N_DEV)
        return aj, g, t % 2

    aj0, g0, p0 = unit_args(0)
    for c in _kv_unit(my, aj0, g0, p0, k_hbm, v_hbm,
                      stage_k, stage_v, copy_sems):
        c.start()

    for t in range(N_DEV * NG):
        h, g = t // NG, t % NG
        aj, _, par = unit_args(t)

        if g == 0 and h < N_DEV - 1:
            rq = pltpu.make_async_remote_copy(
                src_ref=wq_comm.at[h], dst_ref=wq_comm.at[h + 1],
                send_sem=wq_ssem.at[h], recv_sem=wq_rsem.at[h + 1],
                device_id=(right,), device_id_type=pl.DeviceIdType.MESH)
            ro = pltpu.make_async_remote_copy(
                src_ref=wo_comm.at[h], dst_ref=wo_comm.at[h + 1],
                send_sem=wo_ssem.at[h], recv_sem=wo_rsem.at[h + 1],
                device_id=(left,), device_id_type=pl.DeviceIdType.MESH)
            rq.start()
            ro.start()

        for c in _kv_unit(my, aj, g, par, k_hbm, v_hbm,
                          stage_k, stage_v, copy_sems):
            c.wait()
        kg = stage_k[par].astype(jnp.bfloat16)
        vg = stage_v[par].astype(jnp.bfloat16)
        if t + 1 < N_DEV * NG:
            ajn, gn, pn = unit_args(t + 1)
            for c in _kv_unit(my, ajn, gn, pn, k_hbm, v_hbm,
                              stage_k, stage_v, copy_sems):
                c.start()

        qg = jnp.dot(x_ref[g * GS:(g + 1) * GS, :], wq_comm[h],
                     preferred_element_type=jnp.float32).astype(jnp.bfloat16)
        ctx_heads = []
        for hh in range(NH):
            q1 = qg[:, hh * DH:(hh + 1) * DH]
            k1 = kg[:, hh * DH:(hh + 1) * DH]
            v1 = vg[:, hh * DH:(hh + 1) * DH]
            s = lax.dot_general(q1, k1, (((1,), (1,)), ((), ())),
                                preferred_element_type=jnp.float32)
            m = jnp.max(s, axis=1, keepdims=True)
            p = jnp.exp(s - m)
            p = (p / jnp.sum(p, axis=1, keepdims=True)).astype(jnp.bfloat16)
            ctx_heads.append(
                jnp.dot(p, v1, preferred_element_type=jnp.float32)
                .astype(jnp.bfloat16))
        ctx_g = jnp.concatenate(ctx_heads, axis=1)
        rows = pl.ds(g * GS, GS)
        if h == 0:
            out_ref[rows, :] = jnp.dot(
                ctx_g, wo_comm[0], preferred_element_type=jnp.float32)
        elif h == 1:
            ctx_hold[rows, :] = ctx_g
        elif h == 2:
            out_ref[rows, :] += jnp.dot(
                ctx_g, wo_comm[2], preferred_element_type=jnp.float32)
        else:
            out_ref[rows, :] += jnp.dot(
                ctx_g, wo_comm[1], preferred_element_type=jnp.float32)
            out_ref[rows, :] += jnp.dot(
                ctx_hold[rows, :], wo_comm[3],
                preferred_element_type=jnp.float32)

        if g == NG - 1 and h < N_DEV - 1:
            rq.wait()
            ro.wait()


def kernel(x, Wq, K_ext, V_ext, Wo):
    xg = _group_rows((x[0] * SCALE).astype(jnp.bfloat16))
    wq = Wq.astype(jnp.bfloat16)
    wo = Wo.astype(jnp.bfloat16)
    kb = K_ext.reshape(N_DEV, NB, BS, NG * HC)
    vb = V_ext.reshape(N_DEV, NB, BS, NG * HC)

    out = pl.pallas_call(
        _body,
        out_shape=jax.ShapeDtypeStruct((SQ, 1024), jnp.float32),
        in_specs=[pl.BlockSpec(memory_space=pltpu.VMEM)] * 3
        + [pl.BlockSpec(memory_space=pl.ANY)] * 2,
        out_specs=pl.BlockSpec(memory_space=pltpu.VMEM),
        scratch_shapes=[
            pltpu.VMEM((N_DEV, 1024, HC), jnp.bfloat16),
            pltpu.VMEM((N_DEV, 1024, HC), jnp.bfloat16),
            pltpu.VMEM((2, GS, HC), jnp.float32),
            pltpu.VMEM((2, GS, HC), jnp.float32),
            pltpu.VMEM((SQ, HC), jnp.bfloat16),
            pltpu.SemaphoreType.DMA((N_DEV,)),
            pltpu.SemaphoreType.DMA((N_DEV,)),
            pltpu.SemaphoreType.DMA((N_DEV,)),
            pltpu.SemaphoreType.DMA((N_DEV,)),
            pltpu.SemaphoreType.DMA((2, 2)),
        ],
        compiler_params=pltpu.CompilerParams(
            collective_id=0, vmem_limit_bytes=100 * 1024 * 1024),
    )(xg, wq, wo, kb, vb)

    return _ungroup_rows(out)[None].astype(jnp.float32)


# device time: 201264 ns/iter; 2.1291x vs baseline; 2.1291x over previous
import jax
import jax.numpy as jnp
from jax import lax
from jax.experimental import pallas as pl
from jax.experimental.pallas import tpu as pltpu

N_DEV = 4
SQ = 2048
NB = 32
BS = 64
HC = 1024
NH = 8
DH = 128
NG = 4
GS = SQ // NG
SCALE = 0.08838834764831843


def _group_rows(a):
    s = a.shape[0]
    return a.reshape(8, 4, 64, -1).transpose(1, 0, 2, 3).reshape(s, -1)


def _ungroup_rows(a):
    s = a.shape[0]
    return a.reshape(4, 8, 64, -1).transpose(1, 0, 2, 3).reshape(s, -1)


def _kv_unit(aj, g, par, k_hbm, v_hbm, stage_k, stage_v, copy_sems):
    cps = []
    cols = pl.ds(aj * HC, HC)
    for k in range(NH):
        b = NG * k + g
        rows = pl.ds(BS * k, BS)
        cps.append(pltpu.make_async_copy(
            k_hbm.at[b, :, cols], stage_k.at[par, rows],
            copy_sems.at[0, par]))
        cps.append(pltpu.make_async_copy(
            v_hbm.at[b, :, cols], stage_v.at[par, rows],
            copy_sems.at[1, par]))
    return cps


def _body(x_ref, wq_ref, wo_ref, k_hbm, v_hbm, out_ref,
          wq_comm, wo_comm, stage_k, stage_v, ctx_hold,
          wq_ssem, wq_rsem, wo_ssem, wo_rsem, copy_sems):
    my = lax.axis_index("i")
    left = lax.rem(my + N_DEV - 1, N_DEV)
    right = lax.rem(my + 1, N_DEV)

    barrier_sem = pltpu.get_barrier_semaphore()
    for nbr in (left, right):
        pl.semaphore_signal(barrier_sem, inc=1, device_id=(nbr,),
                            device_id_type=pl.DeviceIdType.MESH)
    pl.semaphore_wait(barrier_sem, 2)

    wq_comm[0] = wq_ref[...]
    wo_comm[0] = wo_ref[...]

    def unit_args(t):
        h, g = t // NG, t % NG
        aj = lax.rem(my + N_DEV - h, N_DEV)
        return aj, g, t % 2

    aj0, g0, p0 = unit_args(0)
    for c in _kv_unit(aj0, g0, p0, k_hbm, v_hbm, stage_k, stage_v, copy_sems):
        c.start()

    for t in range(N_DEV * NG):
        h, g = t // NG, t % NG
        aj, _, par = unit_args(t)

        if g == 0 and h < N_DEV - 1:
            rq = pltpu.make_async_remote_copy(
                src_ref=wq_comm.at[h], dst_ref=wq_comm.at[h + 1],
                send_sem=wq_ssem.at[h], recv_sem=wq_rsem.at[h + 1],
                device_id=(right,), device_id_type=pl.DeviceIdType.MESH)
            ro = pltpu.make_async_remote_copy(
                src_ref=wo_comm.at[h], dst_ref=wo_comm.at[h + 1],
                send_sem=wo_ssem.at[h], recv_sem=wo_rsem.at[h + 1],
                device_id=(left,), device_id_type=pl.DeviceIdType.MESH)
            rq.start()
            ro.start()

        for c in _kv_unit(aj, g, par, k_hbm, v_hbm,
                          stage_k, stage_v, copy_sems):
            c.wait()
        if t + 1 < N_DEV * NG:
            ajn, gn, pn = unit_args(t + 1)
            for c in _kv_unit(ajn, gn, pn, k_hbm, v_hbm,
                              stage_k, stage_v, copy_sems):
                c.start()

        qg = jnp.dot(x_ref[g * GS:(g + 1) * GS, :], wq_comm[h],
                     preferred_element_type=jnp.float32).astype(jnp.bfloat16)
        ctx_heads = []
        for hh in range(NH):
            q1 = qg[:, hh * DH:(hh + 1) * DH]
            k1 = stage_k[par, :, hh * DH:(hh + 1) * DH]
            v1 = stage_v[par, :, hh * DH:(hh + 1) * DH]
            s = lax.dot_general(q1, k1, (((1,), (1,)), ((), ())),
                                preferred_element_type=jnp.float32)
            m = jnp.max(s, axis=1, keepdims=True)
            p = jnp.exp(s - m)
            p = (p / jnp.sum(p, axis=1, keepdims=True)).astype(jnp.bfloat16)
            ctx_heads.append(
                jnp.dot(p, v1, preferred_element_type=jnp.float32)
                .astype(jnp.bfloat16))
        ctx_g = jnp.concatenate(ctx_heads, axis=1)
        rows = pl.ds(g * GS, GS)
        if h == 0:
            out_ref[rows, :] = jnp.dot(
                ctx_g, wo_comm[0], preferred_element_type=jnp.float32)
        elif h == 1:
            ctx_hold[rows, :] = ctx_g
        elif h == 2:
            out_ref[rows, :] += jnp.dot(
                ctx_g, wo_comm[2], preferred_element_type=jnp.float32)
        else:
            out_ref[rows, :] += jnp.dot(
                ctx_g, wo_comm[1], preferred_element_type=jnp.float32)
            out_ref[rows, :] += jnp.dot(
                ctx_hold[rows, :], wo_comm[3],
                preferred_element_type=jnp.float32)

        if g == NG - 1 and h < N_DEV - 1:
            rq.wait()
            ro.wait()


def kernel(x, Wq, K_ext, V_ext, Wo):
    my = lax.axis_index("i")
    xg = _group_rows((x[0] * SCALE).astype(jnp.bfloat16))
    wq = Wq.astype(jnp.bfloat16)
    wo = Wo.astype(jnp.bfloat16)
    kb = K_ext[my].astype(jnp.bfloat16).reshape(NB, BS, NG * HC)
    vb = V_ext[my].astype(jnp.bfloat16).reshape(NB, BS, NG * HC)

    out = pl.pallas_call(
        _body,
        out_shape=jax.ShapeDtypeStruct((SQ, 1024), jnp.float32),
        in_specs=[pl.BlockSpec(memory_space=pltpu.VMEM)] * 3
        + [pl.BlockSpec(memory_space=pl.ANY)] * 2,
        out_specs=pl.BlockSpec(memory_space=pltpu.VMEM),
        scratch_shapes=[
            pltpu.VMEM((N_DEV, 1024, HC), jnp.bfloat16),
            pltpu.VMEM((N_DEV, 1024, HC), jnp.bfloat16),
            pltpu.VMEM((2, GS, HC), jnp.bfloat16),
            pltpu.VMEM((2, GS, HC), jnp.bfloat16),
            pltpu.VMEM((SQ, HC), jnp.bfloat16),
            pltpu.SemaphoreType.DMA((N_DEV,)),
            pltpu.SemaphoreType.DMA((N_DEV,)),
            pltpu.SemaphoreType.DMA((N_DEV,)),
            pltpu.SemaphoreType.DMA((N_DEV,)),
            pltpu.SemaphoreType.DMA((2, 2)),
        ],
        compiler_params=pltpu.CompilerParams(
            collective_id=0, vmem_limit_bytes=100 * 1024 * 1024),
    )(xg, wq, wo, kb, vb)

    return _ungroup_rows(out)[None].astype(jnp.float32)


# device time: 193515 ns/iter; 2.2144x vs baseline; 1.0400x over previous
import jax
import jax.numpy as jnp
from jax import lax
from jax.experimental import pallas as pl
from jax.experimental.pallas import tpu as pltpu

N_DEV = 4
SQ = 2048
NB = 32
BS = 64
HC = 1024
NH = 8
DH = 128
NG = 4
GS = SQ // NG
SCALE = 0.08838834764831843


def _group_rows(a):
    s = a.shape[0]
    return a.reshape(8, 4, 64, -1).transpose(1, 0, 2, 3).reshape(s, -1)


def _ungroup_rows(a):
    s = a.shape[0]
    return a.reshape(4, 8, 64, -1).transpose(1, 0, 2, 3).reshape(s, -1)


def _kv_unit(aj, g, par, k_hbm, v_hbm, stage_k, stage_v, copy_sems):
    cps = []
    cols = pl.ds(aj * HC, HC)
    for k in range(NH):
        b = NG * k + g
        rows = pl.ds(BS * k, BS)
        cps.append(pltpu.make_async_copy(
            k_hbm.at[b, :, cols], stage_k.at[par, rows],
            copy_sems.at[0, par]))
        cps.append(pltpu.make_async_copy(
            v_hbm.at[b, :, cols], stage_v.at[par, rows],
            copy_sems.at[1, par]))
    return cps


def _body(x_ref, wq_ref, wo_ref, k_hbm, v_hbm, out_ref,
          wq_comm, wo_comm, stage_k, stage_v, ctx_hold,
          wq_ssem, wq_rsem, wo_ssem, wo_rsem, copy_sems):
    my = lax.axis_index("i")
    left = lax.rem(my + N_DEV - 1, N_DEV)
    right = lax.rem(my + 1, N_DEV)

    barrier_sem = pltpu.get_barrier_semaphore()
    for nbr in (left, right):
        pl.semaphore_signal(barrier_sem, inc=1, device_id=(nbr,),
                            device_id_type=pl.DeviceIdType.MESH)
    pl.semaphore_wait(barrier_sem, 2)

    wq_comm[0] = wq_ref[...]
    wo_comm[0] = wo_ref[...]

    def unit_args(t):
        h, g = t // NG, t % NG
        aj = lax.rem(my + N_DEV - h, N_DEV)
        return aj, g, t % 2

    aj0, g0, p0 = unit_args(0)
    for c in _kv_unit(aj0, g0, p0, k_hbm, v_hbm, stage_k, stage_v, copy_sems):
        c.start()

    for t in range(N_DEV * NG):
        h, g = t // NG, t % NG
        aj, _, par = unit_args(t)

        if g == 0 and h < N_DEV - 1:
            rq = pltpu.make_async_remote_copy(
                src_ref=wq_comm.at[h], dst_ref=wq_comm.at[h + 1],
                send_sem=wq_ssem.at[h], recv_sem=wq_rsem.at[h + 1],
                device_id=(right,), device_id_type=pl.DeviceIdType.MESH)
            ro = pltpu.make_async_remote_copy(
                src_ref=wo_comm.at[h], dst_ref=wo_comm.at[h + 1],
                send_sem=wo_ssem.at[h], recv_sem=wo_rsem.at[h + 1],
                device_id=(left,), device_id_type=pl.DeviceIdType.MESH)
            rq.start()
            ro.start()

        for c in _kv_unit(aj, g, par, k_hbm, v_hbm,
                          stage_k, stage_v, copy_sems):
            c.wait()
        if t + 1 < N_DEV * NG:
            ajn, gn, pn = unit_args(t + 1)
            for c in _kv_unit(ajn, gn, pn, k_hbm, v_hbm,
                              stage_k, stage_v, copy_sems):
                c.start()

        qg = jnp.dot(x_ref[g * GS:(g + 1) * GS, :], wq_comm[h],
                     preferred_element_type=jnp.float32).astype(jnp.bfloat16)
        ctx_heads = []
        for hh in range(NH):
            q1 = qg[:, hh * DH:(hh + 1) * DH]
            k1 = stage_k[par, :, hh * DH:(hh + 1) * DH]
            v1 = stage_v[par, :, hh * DH:(hh + 1) * DH]
            s = lax.dot_general(q1, k1, (((1,), (1,)), ((), ())),
                                preferred_element_type=jnp.float32)
            p = jnp.exp(s)
            denom = jnp.sum(p, axis=1, keepdims=True)
            ctx1 = jnp.dot(p.astype(jnp.bfloat16), v1,
                           preferred_element_type=jnp.float32)
            ctx_heads.append((ctx1 / denom).astype(jnp.bfloat16))
        ctx_g = jnp.concatenate(ctx_heads, axis=1)
        rows = pl.ds(g * GS, GS)
        if h == 0:
            out_ref[rows, :] = jnp.dot(
                ctx_g, wo_comm[0], preferred_element_type=jnp.float32)
        elif h == 1:
            ctx_hold[rows, :] = ctx_g
        elif h == 2:
            out_ref[rows, :] += jnp.dot(
                ctx_g, wo_comm[2], preferred_element_type=jnp.float32)
        else:
            out_ref[rows, :] += jnp.dot(
                ctx_g, wo_comm[1], preferred_element_type=jnp.float32)
            out_ref[rows, :] += jnp.dot(
                ctx_hold[rows, :], wo_comm[3],
                preferred_element_type=jnp.float32)

        if g == NG - 1 and h < N_DEV - 1:
            rq.wait()
            ro.wait()


def kernel(x, Wq, K_ext, V_ext, Wo):
    my = lax.axis_index("i")
    xg = _group_rows((x[0] * SCALE).astype(jnp.bfloat16))
    wq = Wq.astype(jnp.bfloat16)
    wo = Wo.astype(jnp.bfloat16)
    kb = K_ext[my].astype(jnp.bfloat16).reshape(NB, BS, NG * HC)
    vb = V_ext[my].astype(jnp.bfloat16).reshape(NB, BS, NG * HC)

    out = pl.pallas_call(
        _body,
        out_shape=jax.ShapeDtypeStruct((SQ, 1024), jnp.float32),
        in_specs=[pl.BlockSpec(memory_space=pltpu.VMEM)] * 3
        + [pl.BlockSpec(memory_space=pl.ANY)] * 2,
        out_specs=pl.BlockSpec(memory_space=pltpu.VMEM),
        scratch_shapes=[
            pltpu.VMEM((N_DEV, 1024, HC), jnp.bfloat16),
            pltpu.VMEM((N_DEV, 1024, HC), jnp.bfloat16),
            pltpu.VMEM((2, GS, HC), jnp.bfloat16),
            pltpu.VMEM((2, GS, HC), jnp.bfloat16),
            pltpu.VMEM((SQ, HC), jnp.bfloat16),
            pltpu.SemaphoreType.DMA((N_DEV,)),
            pltpu.SemaphoreType.DMA((N_DEV,)),
            pltpu.SemaphoreType.DMA((N_DEV,)),
            pltpu.SemaphoreType.DMA((N_DEV,)),
            pltpu.SemaphoreType.DMA((2, 2)),
        ],
        compiler_params=pltpu.CompilerParams(
            collective_id=0, vmem_limit_bytes=100 * 1024 * 1024),
    )(xg, wq, wo, kb, vb)

    return _ungroup_rows(out)[None].astype(jnp.float32)


# device time: 192997 ns/iter; 2.2203x vs baseline; 1.0027x over previous
import jax
import jax.numpy as jnp
from jax import lax
from jax.experimental import pallas as pl
from jax.experimental.pallas import tpu as pltpu

N_DEV = 4
SQ = 2048
NB = 32
BS = 64
HC = 1024
NH = 8
DH = 128
NG = 4
GS = SQ // NG
SCALE = 0.08838834764831843


def _group_rows(a):
    s = a.shape[0]
    return a.reshape(8, 4, 64, -1).transpose(1, 0, 2, 3).reshape(s, -1)


def _ungroup_rows(a):
    s = a.shape[0]
    return a.reshape(4, 8, 64, -1).transpose(1, 0, 2, 3).reshape(s, -1)


def _kv_unit(aj, g, par, k_hbm, v_hbm, stage_k, stage_v, copy_sems):
    cps = []
    cols = pl.ds(aj * HC, HC)
    for k in range(NH):
        b = NG * k + g
        rows = pl.ds(BS * k, BS)
        cps.append(pltpu.make_async_copy(
            k_hbm.at[b, :, cols], stage_k.at[par, rows],
            copy_sems.at[0, par]))
        cps.append(pltpu.make_async_copy(
            v_hbm.at[b, :, cols], stage_v.at[par, rows],
            copy_sems.at[1, par]))
    return cps


def _body(x_ref, wq_ref, wo_ref, k_hbm, v_hbm, out_ref,
          wq_comm, wo_comm, stage_k, stage_v, ctx_hold,
          wq_ssem, wq_rsem, wo_ssem, wo_rsem, copy_sems):
    my = lax.axis_index("i")
    left = lax.rem(my + N_DEV - 1, N_DEV)
    right = lax.rem(my + 1, N_DEV)

    barrier_sem = pltpu.get_barrier_semaphore()
    for nbr in (left, right):
        pl.semaphore_signal(barrier_sem, inc=1, device_id=(nbr,),
                            device_id_type=pl.DeviceIdType.MESH)
    pl.semaphore_wait(barrier_sem, 2)

    wq_comm[0] = wq_ref[...]
    wo_comm[0] = wo_ref[...]

    def unit_args(t):
        h, g = t // NG, t % NG
        aj = lax.rem(my + N_DEV - h, N_DEV)
        return aj, g, t % 2

    aj0, g0, p0 = unit_args(0)
    for c in _kv_unit(aj0, g0, p0, k_hbm, v_hbm, stage_k, stage_v, copy_sems):
        c.start()

    for t in range(N_DEV * NG):
        h, g = t // NG, t % NG
        aj, _, par = unit_args(t)

        if g == 0 and h < N_DEV - 1:
            rq = pltpu.make_async_remote_copy(
                src_ref=wq_comm.at[h], dst_ref=wq_comm.at[h + 1],
                send_sem=wq_ssem.at[h], recv_sem=wq_rsem.at[h + 1],
                device_id=(right,), device_id_type=pl.DeviceIdType.MESH)
            ro = pltpu.make_async_remote_copy(
                src_ref=wo_comm.at[h], dst_ref=wo_comm.at[h + 1],
                send_sem=wo_ssem.at[h], recv_sem=wo_rsem.at[h + 1],
                device_id=(left,), device_id_type=pl.DeviceIdType.MESH)
            rq.start()
            ro.start()

        for c in _kv_unit(aj, g, par, k_hbm, v_hbm,
                          stage_k, stage_v, copy_sems):
            c.wait()
        if t + 1 < N_DEV * NG:
            ajn, gn, pn = unit_args(t + 1)
            for c in _kv_unit(ajn, gn, pn, k_hbm, v_hbm,
                              stage_k, stage_v, copy_sems):
                c.start()

        if g == 0:
            qh = jnp.dot(x_ref[...], wq_comm[h],
                         preferred_element_type=jnp.float32
                         ).astype(jnp.bfloat16)
        qg = qh[g * GS:(g + 1) * GS, :]

        def _qk(hh):
            q1 = qg[:, hh * DH:(hh + 1) * DH]
            k1 = stage_k[par, :, hh * DH:(hh + 1) * DH]
            return lax.dot_general(q1, k1, (((1,), (1,)), ((), ())),
                                   preferred_element_type=jnp.float32)

        ctx_heads = []
        s_cur = _qk(0)
        for hh in range(NH):
            s_next = _qk(hh + 1) if hh + 1 < NH else None
            v1 = stage_v[par, :, hh * DH:(hh + 1) * DH]
            p = jnp.exp(s_cur)
            denom = jnp.sum(p, axis=1, keepdims=True)
            ctx1 = jnp.dot(p.astype(jnp.bfloat16), v1,
                           preferred_element_type=jnp.float32)
            ctx_heads.append((ctx1 / denom).astype(jnp.bfloat16))
            s_cur = s_next
        ctx_g = jnp.concatenate(ctx_heads, axis=1)
        rows = pl.ds(g * GS, GS)
        if h == 0:
            out_ref[rows, :] = jnp.dot(
                ctx_g, wo_comm[0], preferred_element_type=jnp.float32)
        elif h == 1:
            ctx_hold[rows, :] = ctx_g
        elif h == 2:
            out_ref[rows, :] += jnp.dot(
                ctx_g, wo_comm[2], preferred_element_type=jnp.float32)
        else:
            out_ref[rows, :] += jnp.dot(
                ctx_g, wo_comm[1], preferred_element_type=jnp.float32)
            out_ref[rows, :] += jnp.dot(
                ctx_hold[rows, :], wo_comm[3],
                preferred_element_type=jnp.float32)

        if g == NG - 1 and h < N_DEV - 1:
            rq.wait()
            ro.wait()


def kernel(x, Wq, K_ext, V_ext, Wo):
    my = lax.axis_index("i")
    xg = _group_rows((x[0] * SCALE).astype(jnp.bfloat16))
    wq = Wq.astype(jnp.bfloat16)
    wo = Wo.astype(jnp.bfloat16)
    kb = K_ext[my].astype(jnp.bfloat16).reshape(NB, BS, NG * HC)
    vb = V_ext[my].astype(jnp.bfloat16).reshape(NB, BS, NG * HC)

    out = pl.pallas_call(
        _body,
        out_shape=jax.ShapeDtypeStruct((SQ, 1024), jnp.float32),
        in_specs=[pl.BlockSpec(memory_space=pltpu.VMEM)] * 3
        + [pl.BlockSpec(memory_space=pl.ANY)] * 2,
        out_specs=pl.BlockSpec(memory_space=pltpu.VMEM),
        scratch_shapes=[
            pltpu.VMEM((N_DEV, 1024, HC), jnp.bfloat16),
            pltpu.VMEM((N_DEV, 1024, HC), jnp.bfloat16),
            pltpu.VMEM((2, GS, HC), jnp.bfloat16),
            pltpu.VMEM((2, GS, HC), jnp.bfloat16),
            pltpu.VMEM((SQ, HC), jnp.bfloat16),
            pltpu.SemaphoreType.DMA((N_DEV,)),
            pltpu.SemaphoreType.DMA((N_DEV,)),
            pltpu.SemaphoreType.DMA((N_DEV,)),
            pltpu.SemaphoreType.DMA((N_DEV,)),
            pltpu.SemaphoreType.DMA((2, 2)),
        ],
        compiler_params=pltpu.CompilerParams(
            collective_id=0, vmem_limit_bytes=100 * 1024 * 1024),
    )(xg, wq, wo, kb, vb)

    return _ungroup_rows(out)[None].astype(jnp.float32)
